# baseline (speedup 1.0000x reference)
import jax
import jax.numpy as jnp
from jax import lax
from jax.experimental import pallas as pl
from jax.experimental.pallas import tpu as pltpu
from jax.experimental.pallas import tpu_sc as plsc

B, L = 1024, 50
V2 = 100002
DIM = 32
N_CONT = 4
NC, NS = 2, 16
NPAIR = L // 2


def _div_const(rv, d):
    if d & (d - 1) == 0:
        return lax.shift_right_logical(rv, d.bit_length() - 1)
    m = (1 << 18) // d + 1
    return lax.shift_right_logical(rv * m, 18)


def _gather_chunk(col_v, idxc_v, stage_v, lc):
    @plsc.parallel_loop(0, B // 16, unroll=2)
    def _(t):
        cc = t * 16
        for r in range(lc):
            iv = idxc_v[r, pl.ds(cc, 16)]
            stage_v[r, pl.ds(cc, 16)] = plsc.load_gather(col_v, [iv + 1])


def _cont_unit(x_h, out_h, F, d, u, w4_v, b4_v, xc_v, stage_v, sem_o):
    i = _div_const(u, 6)
    o = u - i * 6
    pltpu.sync_copy(x_h.at[i, pl.ds(o * 8, 8), :], xc_v)
    wv = w4_v[i, pl.ds(0, 16)]
    bv = b4_v[i, pl.ds(0, 16)]

    def body(s, c):
        for dl in range(8):
            xv = xc_v[dl, pl.ds(s * 16, 16)]
            stage_v[dl, pl.ds(s * 16, 16)] = xv * wv + bv
        return c
    lax.fori_loop(0, B // 16, body, 0)
    return pltpu.async_copy(
        stage_v, out_h.at[pl.ds(o * 8, 8), F + i, d, :], sem_o)


def _group(idx_h, tab_h, x_h, w_h, b_h, out_h, F, d, ch_layout, iota,
           col_v, idxc_v, stage_a, stage_b, w_v, b_v, w4_v, b4_v,
           sem_c, sem_o):
    pltpu.sync_copy(w_h, w_v)
    pltpu.sync_copy(b_h, b_v)
    izero = iota * 0
    dv = izero + d
    for i in range(N_CONT):
        w4_v[i, pl.ds(0, 16)] = plsc.load_gather(w_v, [izero + i, dv])
        b4_v[i, pl.ds(0, 16)] = plsc.load_gather(b_v, [izero + i, dv])
    xc_v = idxc_v.bitcast(jnp.float32)
    upf = 24 // F

    def per_feature(f, c):
        hcol = pltpu.async_copy(tab_h.at[f, d, :], col_v, sem_c)

        stages = (stage_a, stage_b)
        houts = [None, None]
        slot = 0
        for k in range(upf):
            stg = stages[slot % 2]
            _ = houts[slot % 2].wait() if houts[slot % 2] is not None else None
            houts[slot % 2] = _cont_unit(
                x_h, out_h, F, d, f * upf + k, w4_v, b4_v, xc_v, stg, sem_o)
            slot += 1

        hcol.wait()

        for cj in range(6):
            stg = stages[slot % 2]
            if houts[slot % 2] is not None:
                houts[slot % 2].wait()
            if ch_layout:
                pltpu.sync_copy(idx_h.at[pl.ds(cj * 8, 8), f, :], idxc_v)
            else:
                pltpu.sync_copy(idx_h.at[f, pl.ds(cj * 8, 8), :], idxc_v)
            _gather_chunk(col_v, idxc_v, stg, 8)
            houts[slot % 2] = pltpu.async_copy(
                stg, out_h.at[pl.ds(cj * 8, 8), f, d, :], sem_o)
            slot += 1
        houts[0].wait()
        houts[1].wait()
        if ch_layout:
            pltpu.sync_copy(idx_h.at[pl.ds(48, 2), f, :],
                            idxc_v.at[pl.ds(0, 2), :])
        else:
            pltpu.sync_copy(idx_h.at[f, pl.ds(48, 2), :],
                            idxc_v.at[pl.ds(0, 2), :])
        _gather_chunk(col_v, idxc_v, stage_a, 2)
        pltpu.sync_copy(stage_a.at[pl.ds(0, 2), :],
                        out_h.at[pl.ds(48, 2), f, d, :])
        return c
    lax.fori_loop(0, F, per_feature, 0)

    for i in range(N_CONT):
        pltpu.sync_copy(x_h.at[i, pl.ds(48, 2), :],
                        xc_v.at[pl.ds(0, 2), :])
        wv = w4_v[i, pl.ds(0, 16)]
        bv = b4_v[i, pl.ds(0, 16)]

        def tail_body(s, c, i=i, wv=wv, bv=bv):
            for dl in range(2):
                xv = xc_v[dl, pl.ds(s * 16, 16)]
                stage_a[dl, pl.ds(s * 16, 16)] = xv * wv + bv
            return c
        lax.fori_loop(0, B // 16, tail_body, 0)
        pltpu.sync_copy(stage_a.at[pl.ds(0, 2), :],
                        out_h.at[pl.ds(48, 2), F + i, d, :])


def _sc_body(idx_q, idx_c, idx_f, x_q, x_c, x_f, tab_q, tab_c, tab_f,
             w_q, b_q, w_c, b_c, w_f, b_f, out_q, out_c, out_f,
             col_v, idxc_v, stage_a, stage_b, w_v, b_v, w4_v, b4_v,
             sem_c, sem_o):
    d = lax.axis_index("s") * NC + lax.axis_index("c")
    iota = lax.iota(jnp.int32, 16)
    rest = (col_v, idxc_v, stage_a, stage_b, w_v, b_v, w4_v, b4_v,
            sem_c, sem_o)
    _group(idx_q, tab_q, x_q, w_q, b_q, out_q, 6, d, False, iota, *rest)
    _group(idx_c, tab_c, x_c, w_c, b_c, out_c, 8, d, True, iota, *rest)
    _group(idx_f, tab_f, x_f, w_f, b_f, out_f, 6, d, False, iota, *rest)


_sc_kernel = pl.kernel(
    _sc_body,
    out_type=[
        jax.ShapeDtypeStruct((L, 10, DIM, B), jnp.float32),
        jax.ShapeDtypeStruct((L, 12, DIM, B), jnp.float32),
        jax.ShapeDtypeStruct((L, 10, DIM, B), jnp.float32),
    ],
    mesh=plsc.VectorSubcoreMesh(
        core_axis_name="c", subcore_axis_name="s",
        num_cores=NC, num_subcores=NS),
    scratch_types=[
        pltpu.VMEM((V2,), jnp.float32),
        pltpu.VMEM((8, B), jnp.int32),
        pltpu.VMEM((8, B), jnp.float32),
        pltpu.VMEM((8, B), jnp.float32),
        pltpu.VMEM((N_CONT, DIM), jnp.float32),
        pltpu.VMEM((N_CONT, DIM), jnp.float32),
        pltpu.VMEM((N_CONT, 16), jnp.float32),
        pltpu.VMEM((N_CONT, 16), jnp.float32),
        pltpu.SemaphoreType.DMA,
        pltpu.SemaphoreType.DMA,
    ],
    compiler_params=pltpu.CompilerParams(needs_layout_passes=False),
)


@jax.jit
def kernel(batch_feature_tensor_pay_QOE_discrete,
           batch_feature_tensor_pay_CHONGHE_discrete,
           batch_feature_tensor_pay_FUFEI_discrete,
           batch_feature_tensor_pay_QOE_continue,
           batch_feature_tensor_pay_CHONGHE_continue,
           batch_feature_tensor_pay_FUFEI_continue,
           QOE_tables, CHONGHE_tables, FUFEI_tables,
           W_QOE, b_QOE, W_CHONGHE, b_CHONGHE, W_FUFEI, b_FUFEI):
    idx_q = batch_feature_tensor_pay_QOE_discrete.astype(jnp.int32).transpose(2, 1, 0)
    idx_c = batch_feature_tensor_pay_CHONGHE_discrete.astype(jnp.int32).transpose(1, 2, 0)
    idx_f = batch_feature_tensor_pay_FUFEI_discrete.astype(jnp.int32).transpose(2, 1, 0)
    x_q = batch_feature_tensor_pay_QOE_continue.astype(jnp.float32).transpose(2, 1, 0)
    x_c = batch_feature_tensor_pay_CHONGHE_continue.astype(jnp.float32).transpose(2, 1, 0)
    x_f = batch_feature_tensor_pay_FUFEI_continue.astype(jnp.float32).transpose(2, 1, 0)
    tab_q = QOE_tables.transpose(0, 2, 1)
    tab_c = CHONGHE_tables.transpose(0, 2, 1)
    tab_f = FUFEI_tables.transpose(0, 2, 1)
    out_q, out_c, out_f = _sc_kernel(
        idx_q, idx_c, idx_f, x_q, x_c, x_f, tab_q, tab_c, tab_f,
        W_QOE, b_QOE, W_CHONGHE, b_CHONGHE, W_FUFEI, b_FUFEI)
    return (out_q.transpose(3, 0, 1, 2),
            out_c.transpose(3, 0, 1, 2),
            out_f.transpose(3, 0, 1, 2))


# speedup vs baseline: 1.0223x; 1.0223x over previous
import jax
import jax.numpy as jnp
from jax import lax
from jax.experimental import pallas as pl
from jax.experimental.pallas import tpu as pltpu
from jax.experimental.pallas import tpu_sc as plsc

B, L = 1024, 50
V2 = 100002
DIM = 32
N_CONT = 4
NC, NS = 2, 16
NPAIR = L // 2


def _div_const(rv, d):
    if d & (d - 1) == 0:
        return lax.shift_right_logical(rv, d.bit_length() - 1)
    m = (1 << 18) // d + 1
    return lax.shift_right_logical(rv * m, 18)


def _gather_chunk(col_v, idxc_v, stage_v, lc):
    @plsc.parallel_loop(0, (lc * 1024) // 16, unroll=8)
    def _(s):
        r = lax.shift_right_logical(s, 6)
        cc = (s & 63) * 16
        iv = idxc_v[r, pl.ds(cc, 16)]
        stage_v[r, pl.ds(cc, 16)] = plsc.load_gather(col_v, [iv + 1])


def _cont_unit(x_h, out_h, F, d, u, w4_v, b4_v, xc_v, stage_v, sem_o):
    i = _div_const(u, 6)
    o = u - i * 6
    pltpu.sync_copy(x_h.at[i, pl.ds(o * 8, 8), :], xc_v)
    wv = w4_v[i, pl.ds(0, 16)]
    bv = b4_v[i, pl.ds(0, 16)]

    def body(s, c):
        for dl in range(8):
            xv = xc_v[dl, pl.ds(s * 16, 16)]
            stage_v[dl, pl.ds(s * 16, 16)] = xv * wv + bv
        return c
    lax.fori_loop(0, B // 16, body, 0)
    return pltpu.async_copy(
        stage_v, out_h.at[pl.ds(o * 8, 8), F + i, d, :], sem_o)


def _group(idx_h, tab_h, x_h, w_h, b_h, out_h, F, d, ch_layout, iota,
           col_v, idxc_v, stage_a, stage_b, w_v, b_v, w4_v, b4_v,
           sem_c, sem_o):
    pltpu.sync_copy(w_h, w_v)
    pltpu.sync_copy(b_h, b_v)
    izero = iota * 0
    dv = izero + d
    for i in range(N_CONT):
        w4_v[i, pl.ds(0, 16)] = plsc.load_gather(w_v, [izero + i, dv])
        b4_v[i, pl.ds(0, 16)] = plsc.load_gather(b_v, [izero + i, dv])
    xc_v = idxc_v.bitcast(jnp.float32)
    upf = 24 // F

    def per_feature(f, c):
        hcol = pltpu.async_copy(tab_h.at[f, d, :], col_v, sem_c)

        stages = (stage_a, stage_b)
        houts = [None, None]
        slot = 0
        for k in range(upf):
            stg = stages[slot % 2]
            _ = houts[slot % 2].wait() if houts[slot % 2] is not None else None
            houts[slot % 2] = _cont_unit(
                x_h, out_h, F, d, f * upf + k, w4_v, b4_v, xc_v, stg, sem_o)
            slot += 1

        hcol.wait()

        for cj in range(6):
            stg = stages[slot % 2]
            if houts[slot % 2] is not None:
                houts[slot % 2].wait()
            if ch_layout:
                pltpu.sync_copy(idx_h.at[pl.ds(cj * 8, 8), f, :], idxc_v)
            else:
                pltpu.sync_copy(idx_h.at[f, pl.ds(cj * 8, 8), :], idxc_v)
            _gather_chunk(col_v, idxc_v, stg, 8)
            houts[slot % 2] = pltpu.async_copy(
                stg, out_h.at[pl.ds(cj * 8, 8), f, d, :], sem_o)
            slot += 1
        houts[0].wait()
        houts[1].wait()
        if ch_layout:
            pltpu.sync_copy(idx_h.at[pl.ds(48, 2), f, :],
                            idxc_v.at[pl.ds(0, 2), :])
        else:
            pltpu.sync_copy(idx_h.at[f, pl.ds(48, 2), :],
                            idxc_v.at[pl.ds(0, 2), :])
        _gather_chunk(col_v, idxc_v, stage_a, 2)
        pltpu.sync_copy(stage_a.at[pl.ds(0, 2), :],
                        out_h.at[pl.ds(48, 2), f, d, :])
        return c
    lax.fori_loop(0, F, per_feature, 0)

    for i in range(N_CONT):
        pltpu.sync_copy(x_h.at[i, pl.ds(48, 2), :],
                        xc_v.at[pl.ds(0, 2), :])
        wv = w4_v[i, pl.ds(0, 16)]
        bv = b4_v[i, pl.ds(0, 16)]

        def tail_body(s, c, i=i, wv=wv, bv=bv):
            for dl in range(2):
                xv = xc_v[dl, pl.ds(s * 16, 16)]
                stage_a[dl, pl.ds(s * 16, 16)] = xv * wv + bv
            return c
        lax.fori_loop(0, B // 16, tail_body, 0)
        pltpu.sync_copy(stage_a.at[pl.ds(0, 2), :],
                        out_h.at[pl.ds(48, 2), F + i, d, :])


def _sc_body(idx_q, idx_c, idx_f, x_q, x_c, x_f, tab_q, tab_c, tab_f,
             w_q, b_q, w_c, b_c, w_f, b_f, out_q, out_c, out_f,
             col_v, idxc_v, stage_a, stage_b, w_v, b_v, w4_v, b4_v,
             sem_c, sem_o):
    d = lax.axis_index("s") * NC + lax.axis_index("c")
    iota = lax.iota(jnp.int32, 16)
    rest = (col_v, idxc_v, stage_a, stage_b, w_v, b_v, w4_v, b4_v,
            sem_c, sem_o)
    _group(idx_q, tab_q, x_q, w_q, b_q, out_q, 6, d, False, iota, *rest)
    _group(idx_c, tab_c, x_c, w_c, b_c, out_c, 8, d, True, iota, *rest)
    _group(idx_f, tab_f, x_f, w_f, b_f, out_f, 6, d, False, iota, *rest)


_sc_kernel = pl.kernel(
    _sc_body,
    out_type=[
        jax.ShapeDtypeStruct((L, 10, DIM, B), jnp.float32),
        jax.ShapeDtypeStruct((L, 12, DIM, B), jnp.float32),
        jax.ShapeDtypeStruct((L, 10, DIM, B), jnp.float32),
    ],
    mesh=plsc.VectorSubcoreMesh(
        core_axis_name="c", subcore_axis_name="s",
        num_cores=NC, num_subcores=NS),
    scratch_types=[
        pltpu.VMEM((V2,), jnp.float32),
        pltpu.VMEM((8, B), jnp.int32),
        pltpu.VMEM((8, B), jnp.float32),
        pltpu.VMEM((8, B), jnp.float32),
        pltpu.VMEM((N_CONT, DIM), jnp.float32),
        pltpu.VMEM((N_CONT, DIM), jnp.float32),
        pltpu.VMEM((N_CONT, 16), jnp.float32),
        pltpu.VMEM((N_CONT, 16), jnp.float32),
        pltpu.SemaphoreType.DMA,
        pltpu.SemaphoreType.DMA,
    ],
    compiler_params=pltpu.CompilerParams(needs_layout_passes=False),
)


@jax.jit
def kernel(batch_feature_tensor_pay_QOE_discrete,
           batch_feature_tensor_pay_CHONGHE_discrete,
           batch_feature_tensor_pay_FUFEI_discrete,
           batch_feature_tensor_pay_QOE_continue,
           batch_feature_tensor_pay_CHONGHE_continue,
           batch_feature_tensor_pay_FUFEI_continue,
           QOE_tables, CHONGHE_tables, FUFEI_tables,
           W_QOE, b_QOE, W_CHONGHE, b_CHONGHE, W_FUFEI, b_FUFEI):
    idx_q = batch_feature_tensor_pay_QOE_discrete.astype(jnp.int32).transpose(2, 1, 0)
    idx_c = batch_feature_tensor_pay_CHONGHE_discrete.astype(jnp.int32).transpose(1, 2, 0)
    idx_f = batch_feature_tensor_pay_FUFEI_discrete.astype(jnp.int32).transpose(2, 1, 0)
    x_q = batch_feature_tensor_pay_QOE_continue.astype(jnp.float32).transpose(2, 1, 0)
    x_c = batch_feature_tensor_pay_CHONGHE_continue.astype(jnp.float32).transpose(2, 1, 0)
    x_f = batch_feature_tensor_pay_FUFEI_continue.astype(jnp.float32).transpose(2, 1, 0)
    tab_q = QOE_tables.transpose(0, 2, 1)
    tab_c = CHONGHE_tables.transpose(0, 2, 1)
    tab_f = FUFEI_tables.transpose(0, 2, 1)
    out_q, out_c, out_f = _sc_kernel(
        idx_q, idx_c, idx_f, x_q, x_c, x_f, tab_q, tab_c, tab_f,
        W_QOE, b_QOE, W_CHONGHE, b_CHONGHE, W_FUFEI, b_FUFEI)
    return (out_q.transpose(3, 0, 1, 2),
            out_c.transpose(3, 0, 1, 2),
            out_f.transpose(3, 0, 1, 2))
